# baseline (speedup 1.0000x reference)
import functools

import jax
import jax.numpy as jnp
from jax import lax
from jax.experimental import pallas as pl
from jax.experimental.pallas import tpu as pltpu
from jax.experimental.pallas import tpu_sc as plsc

N_TASKS = 10
FILTER_NUM = 128
NHID = 2048
BATCH = 4096

NC = 2
NS = 16
L = 16
NW = NC * NS
B_PER_W = BATCH // NW
F_CHUNK = 32


def _sigmoid_block(tstage, sv, rows, width):
    def body(r, _):
        for j in range(width // L):
            x = tstage[r, pl.ds(j * L, L)]
            z = sv * x
            ez = jnp.exp(-jnp.abs(z))
            d = 1.0 / (1.0 + ez)
            tstage[r, pl.ds(j * L, L)] = jnp.where(z >= 0, d, ez * d)
        return 0
    lax.fori_loop(0, rows, body, 0)


def _sc_kernel(t_ref, s_ref, ec1, ec2, ec3, ef1, ef2, ef3,
               oc1, oc2, oc3, of1, of2, of3,
               sgc1, sgc2, sgc3, sgf1, sgf2, sgf3,
               svec_v, idx_v, tstage, rows_c, rows_f, sem):
    cid = lax.axis_index("c")
    sid = lax.axis_index("s")

    pltpu.sync_copy(s_ref, svec_v)
    sv = svec_v[...]

    col0 = sid * FILTER_NUM
    for src, dst in ((ef1, sgf1), (ef2, sgf2), (ef3, sgf3)):
        pltpu.sync_copy(src.at[:, pl.ds(col0, FILTER_NUM)], tstage)
        _sigmoid_block(tstage, sv, N_TASKS, FILTER_NUM)
        pltpu.sync_copy(tstage, dst.at[:, pl.ds(col0, FILTER_NUM)])
    for i, (src, dst) in enumerate(((ec1, sgc1), (ec2, sgc2), (ec3, sgc3))):
        @pl.when(sid == i)
        def _():
            pltpu.sync_copy(src, tstage)
            _sigmoid_block(tstage, sv, N_TASKS, FILTER_NUM)
            pltpu.sync_copy(tstage, dst)

    plsc.subcore_barrier()

    wid = sid * NC + cid
    base = wid * B_PER_W
    pltpu.sync_copy(t_ref.at[pl.ds(base, B_PER_W)], idx_v)

    for spm, out in ((sgc1, oc1), (sgc2, oc2), (sgc3, oc3)):
        pltpu.async_copy(spm.at[idx_v], rows_c, sem).wait()
        pltpu.sync_copy(rows_c, out.at[pl.ds(base, B_PER_W)])

    for spm, out in ((sgf1, of1), (sgf2, of2), (sgf3, of3)):
        for k in range(B_PER_W // F_CHUNK):
            idx_k = idx_v.at[pl.ds(k * F_CHUNK, F_CHUNK)]
            pltpu.async_copy(spm.at[idx_k], rows_f, sem).wait()
            pltpu.sync_copy(rows_f, out.at[pl.ds(base + k * F_CHUNK, F_CHUNK)])


@jax.jit
def _run(t, svec, ec1, ec2, ec3, efc1, efc2, efc3):
    f32 = jnp.float32
    call = functools.partial(
        pl.kernel,
        out_type=[
            jax.ShapeDtypeStruct((BATCH, FILTER_NUM), f32),
            jax.ShapeDtypeStruct((BATCH, FILTER_NUM), f32),
            jax.ShapeDtypeStruct((BATCH, FILTER_NUM), f32),
            jax.ShapeDtypeStruct((BATCH, NHID), f32),
            jax.ShapeDtypeStruct((BATCH, NHID), f32),
            jax.ShapeDtypeStruct((BATCH, NHID), f32),
        ],
        mesh=plsc.VectorSubcoreMesh(core_axis_name="c", subcore_axis_name="s"),
        scratch_types=[
            pltpu.HBM((N_TASKS, FILTER_NUM), f32),
            pltpu.HBM((N_TASKS, FILTER_NUM), f32),
            pltpu.HBM((N_TASKS, FILTER_NUM), f32),
            pltpu.HBM((N_TASKS, NHID), f32),
            pltpu.HBM((N_TASKS, NHID), f32),
            pltpu.HBM((N_TASKS, NHID), f32),
            pltpu.VMEM((L,), f32),
            pltpu.VMEM((B_PER_W,), jnp.int32),
            pltpu.VMEM((N_TASKS, FILTER_NUM), f32),
            pltpu.VMEM((B_PER_W, FILTER_NUM), f32),
            pltpu.VMEM((F_CHUNK, NHID), f32),
            pltpu.SemaphoreType.DMA,
        ],
    )(_sc_kernel)
    return tuple(call(t, svec, ec1, ec2, ec3, efc1, efc2, efc3))


def kernel(t, s, ec1, ec2, ec3, efc1, efc2, efc3):
    t32 = t.astype(jnp.int32)
    svec = jnp.full((L,), s, dtype=jnp.float32)
    return _run(t32, svec, ec1, ec2, ec3, efc1, efc2, efc3)


# speedup vs baseline: 3.1107x; 3.1107x over previous
import functools

import jax
import jax.numpy as jnp
from jax import lax
from jax.experimental import pallas as pl
from jax.experimental.pallas import tpu as pltpu
from jax.experimental.pallas import tpu_sc as plsc

N_TASKS = 10
FILTER_NUM = 128
NHID = 2048
BATCH = 4096

NC = 2
NS = 16
L = 16
NW = NC * NS
B_PER_W = BATCH // NW


def _sigmoid_block(tstage, sv, rows, width):
    def body(r, _):
        for j in range(width // L):
            x = tstage[r, pl.ds(j * L, L)]
            z = sv * x
            ez = jnp.exp(-jnp.abs(z))
            d = 1.0 / (1.0 + ez)
            tstage[r, pl.ds(j * L, L)] = jnp.where(z >= 0, d, ez * d)
        return 0
    lax.fori_loop(0, rows, body, 0)


def _sc_kernel(t_ref, s_ref, ec1, ec2, ec3, ef1, ef2, ef3,
               oc1, oc2, oc3, of1, of2, of3,
               sgc1, sgc2, sgc3, sgf1, sgf2, sgf3,
               svec_v, idx_v, tstage,
               tc1, tc2, tc3, tf1, tf2, tf3, sem):
    cid = lax.axis_index("c")
    sid = lax.axis_index("s")

    pltpu.sync_copy(s_ref, svec_v)
    sv = svec_v[...]

    col0 = sid * FILTER_NUM
    for src, dst in ((ef1, sgf1), (ef2, sgf2), (ef3, sgf3)):
        pltpu.sync_copy(src.at[:, pl.ds(col0, FILTER_NUM)], tstage)
        _sigmoid_block(tstage, sv, N_TASKS, FILTER_NUM)
        pltpu.sync_copy(tstage, dst.at[:, pl.ds(col0, FILTER_NUM)])
    for i, (src, dst) in enumerate(((ec1, sgc1), (ec2, sgc2), (ec3, sgc3))):
        @pl.when(sid == i)
        def _():
            pltpu.sync_copy(src, tstage)
            _sigmoid_block(tstage, sv, N_TASKS, FILTER_NUM)
            pltpu.sync_copy(tstage, dst)

    plsc.subcore_barrier()

    for spm, vm in ((sgc1, tc1), (sgc2, tc2), (sgc3, tc3),
                    (sgf1, tf1), (sgf2, tf2), (sgf3, tf3)):
        pltpu.sync_copy(spm, vm)

    wid = sid * NC + cid
    base = wid * B_PER_W
    pltpu.sync_copy(t_ref.at[pl.ds(base, B_PER_W)], idx_v)

    def chunk_body(c, _):
        v = idx_v[pl.ds(c * L, L)]
        for j in range(L):
            r = v[j]
            i = c * L + j
            for tbl, out in ((tc1, oc1), (tc2, oc2), (tc3, oc3),
                             (tf1, of1), (tf2, of2), (tf3, of3)):
                pltpu.async_copy(tbl.at[r], out.at[base + i], sem)
        return 0
    lax.fori_loop(0, B_PER_W // L, chunk_body, 0)

    def drain_body(i, _):
        for tbl, out in ((tc1, oc1), (tc2, oc2), (tc3, oc3),
                         (tf1, of1), (tf2, of2), (tf3, of3)):
            pltpu.make_async_copy(tbl.at[0], out.at[base + i], sem).wait()
        return 0
    lax.fori_loop(0, B_PER_W, drain_body, 0)


@jax.jit
def _run(t, svec, ec1, ec2, ec3, efc1, efc2, efc3):
    f32 = jnp.float32
    call = functools.partial(
        pl.kernel,
        out_type=[
            jax.ShapeDtypeStruct((BATCH, FILTER_NUM), f32),
            jax.ShapeDtypeStruct((BATCH, FILTER_NUM), f32),
            jax.ShapeDtypeStruct((BATCH, FILTER_NUM), f32),
            jax.ShapeDtypeStruct((BATCH, NHID), f32),
            jax.ShapeDtypeStruct((BATCH, NHID), f32),
            jax.ShapeDtypeStruct((BATCH, NHID), f32),
        ],
        mesh=plsc.VectorSubcoreMesh(core_axis_name="c", subcore_axis_name="s"),
        scratch_types=[
            pltpu.HBM((N_TASKS, FILTER_NUM), f32),
            pltpu.HBM((N_TASKS, FILTER_NUM), f32),
            pltpu.HBM((N_TASKS, FILTER_NUM), f32),
            pltpu.HBM((N_TASKS, NHID), f32),
            pltpu.HBM((N_TASKS, NHID), f32),
            pltpu.HBM((N_TASKS, NHID), f32),
            pltpu.VMEM((L,), f32),
            pltpu.VMEM((B_PER_W,), jnp.int32),
            pltpu.VMEM((N_TASKS, FILTER_NUM), f32),
            pltpu.VMEM((N_TASKS, FILTER_NUM), f32),
            pltpu.VMEM((N_TASKS, FILTER_NUM), f32),
            pltpu.VMEM((N_TASKS, FILTER_NUM), f32),
            pltpu.VMEM((N_TASKS, NHID), f32),
            pltpu.VMEM((N_TASKS, NHID), f32),
            pltpu.VMEM((N_TASKS, NHID), f32),
            pltpu.SemaphoreType.DMA,
        ],
    )(_sc_kernel)
    return tuple(call(t, svec, ec1, ec2, ec3, efc1, efc2, efc3))


def kernel(t, s, ec1, ec2, ec3, efc1, efc2, efc3):
    t32 = t.astype(jnp.int32)
    svec = jnp.full((L,), s, dtype=jnp.float32)
    return _run(t32, svec, ec1, ec2, ec3, efc1, efc2, efc3)


# speedup vs baseline: 3.7117x; 1.1932x over previous
import functools

import jax
import jax.numpy as jnp
from jax import lax
from jax.experimental import pallas as pl
from jax.experimental.pallas import tpu as pltpu
from jax.experimental.pallas import tpu_sc as plsc

N_TASKS = 10
FILTER_NUM = 128
NHID = 2048
BATCH = 4096

NC = 2
NS = 16
L = 16
NW = NC * NS
B_PER_W = BATCH // NW

TC_BLOCK = 256


def _sigmoid_block(tstage, sv, rows, width):
    def body(r, _):
        for j in range(width // L):
            x = tstage[r, pl.ds(j * L, L)]
            z = sv * x
            ez = jnp.exp(-jnp.abs(z))
            d = 1.0 / (1.0 + ez)
            tstage[r, pl.ds(j * L, L)] = jnp.where(z >= 0, d, ez * d)
        return 0
    lax.fori_loop(0, rows, body, 0)


def _sc_kernel(t_ref, s_ref, ec1, ec2, ec3, ef1, ef2,
               oc1, oc2, oc3, of1, of2,
               sgc1, sgc2, sgc3, sgf1, sgf2,
               svec_v, idx_v, tstage,
               tc1, tc2, tc3, tf1, tf2, sem):
    cid = lax.axis_index("c")
    sid = lax.axis_index("s")

    pltpu.sync_copy(s_ref, svec_v)
    sv = svec_v[...]

    col0 = sid * FILTER_NUM
    for src, dst in ((ef1, sgf1), (ef2, sgf2)):
        pltpu.sync_copy(src.at[:, pl.ds(col0, FILTER_NUM)], tstage)
        _sigmoid_block(tstage, sv, N_TASKS, FILTER_NUM)
        pltpu.sync_copy(tstage, dst.at[:, pl.ds(col0, FILTER_NUM)])
    for i, (src, dst) in enumerate(((ec1, sgc1), (ec2, sgc2), (ec3, sgc3))):
        @pl.when(sid == i)
        def _():
            pltpu.sync_copy(src, tstage)
            _sigmoid_block(tstage, sv, N_TASKS, FILTER_NUM)
            pltpu.sync_copy(tstage, dst)

    plsc.subcore_barrier()

    for spm, vm in ((sgc1, tc1), (sgc2, tc2), (sgc3, tc3),
                    (sgf1, tf1), (sgf2, tf2)):
        pltpu.sync_copy(spm, vm)

    wid = sid * NC + cid
    base = wid * B_PER_W
    pltpu.sync_copy(t_ref.at[pl.ds(base, B_PER_W)], idx_v)

    pairs = ((tc1, oc1), (tc2, oc2), (tc3, oc3), (tf1, of1), (tf2, of2))

    def chunk_body(c, _):
        v = idx_v[pl.ds(c * L, L)]
        for j in range(L):
            r = v[j]
            i = c * L + j
            for tbl, out in pairs:
                pltpu.async_copy(tbl.at[r], out.at[base + i], sem)
        return 0
    lax.fori_loop(0, B_PER_W // L, chunk_body, 0)

    def drain_body(i, _):
        for tbl, out in pairs:
            pltpu.make_async_copy(tbl.at[0], out.at[base + i], sem).wait()
        return 0
    lax.fori_loop(0, B_PER_W, drain_body, 0)


def _tc_kernel(s_ref, t_ref, tbl_ref, out_ref):
    i = pl.program_id(0)
    s = s_ref[0]
    sig = jax.nn.sigmoid(s * tbl_ref[...])
    tb = t_ref[pl.ds(i * TC_BLOCK, TC_BLOCK)]
    onehot = (tb[:, None] == lax.broadcasted_iota(jnp.int32, (TC_BLOCK, N_TASKS), 1)
              ).astype(jnp.float32)
    out_ref[...] = jnp.dot(onehot, sig, preferred_element_type=jnp.float32)


@jax.jit
def _run(t, svec, ec1, ec2, ec3, efc1, efc2, efc3):
    f32 = jnp.float32
    sc_call = functools.partial(
        pl.kernel,
        out_type=[
            jax.ShapeDtypeStruct((BATCH, FILTER_NUM), f32),
            jax.ShapeDtypeStruct((BATCH, FILTER_NUM), f32),
            jax.ShapeDtypeStruct((BATCH, FILTER_NUM), f32),
            jax.ShapeDtypeStruct((BATCH, NHID), f32),
            jax.ShapeDtypeStruct((BATCH, NHID), f32),
        ],
        mesh=plsc.VectorSubcoreMesh(core_axis_name="c", subcore_axis_name="s"),
        scratch_types=[
            pltpu.HBM((N_TASKS, FILTER_NUM), f32),
            pltpu.HBM((N_TASKS, FILTER_NUM), f32),
            pltpu.HBM((N_TASKS, FILTER_NUM), f32),
            pltpu.HBM((N_TASKS, NHID), f32),
            pltpu.HBM((N_TASKS, NHID), f32),
            pltpu.VMEM((L,), f32),
            pltpu.VMEM((B_PER_W,), jnp.int32),
            pltpu.VMEM((N_TASKS, FILTER_NUM), f32),
            pltpu.VMEM((N_TASKS, FILTER_NUM), f32),
            pltpu.VMEM((N_TASKS, FILTER_NUM), f32),
            pltpu.VMEM((N_TASKS, FILTER_NUM), f32),
            pltpu.VMEM((N_TASKS, NHID), f32),
            pltpu.VMEM((N_TASKS, NHID), f32),
            pltpu.SemaphoreType.DMA,
        ],
    )(_sc_kernel)
    oc1, oc2, oc3, of1, of2 = sc_call(t, svec, ec1, ec2, ec3, efc1, efc2)

    of3 = pl.pallas_call(
        _tc_kernel,
        grid=(BATCH // TC_BLOCK,),
        in_specs=[
            pl.BlockSpec(memory_space=pltpu.SMEM),
            pl.BlockSpec(memory_space=pltpu.VMEM),
            pl.BlockSpec(memory_space=pltpu.VMEM),
        ],
        out_specs=pl.BlockSpec((TC_BLOCK, NHID), lambda i: (i, 0)),
        out_shape=jax.ShapeDtypeStruct((BATCH, NHID), f32),
    )(svec[:1], t, efc3)

    return (oc1, oc2, oc3, of1, of2, of3)


def kernel(t, s, ec1, ec2, ec3, efc1, efc2, efc3):
    t32 = t.astype(jnp.int32)
    svec = jnp.full((L,), s, dtype=jnp.float32)
    return _run(t32, svec, ec1, ec2, ec3, efc1, efc2, efc3)


# speedup vs baseline: 3.9507x; 1.0644x over previous
import functools

import jax
import jax.numpy as jnp
from jax import lax
from jax.experimental import pallas as pl
from jax.experimental.pallas import tpu as pltpu
from jax.experimental.pallas import tpu_sc as plsc

N_TASKS = 10
FILTER_NUM = 128
NHID = 2048
BATCH = 4096

NC = 2
NS = 16
L = 16
NW = NC * NS
B_PER_W = BATCH // NW

TC_BLOCK = 256


def _sigmoid_block(tstage, sv, rows, width):
    def body(r, _):
        for j in range(width // L):
            x = tstage[r, pl.ds(j * L, L)]
            z = sv * x
            ez = jnp.exp(-jnp.abs(z))
            d = 1.0 / (1.0 + ez)
            tstage[r, pl.ds(j * L, L)] = jnp.where(z >= 0, d, ez * d)
        return 0
    lax.fori_loop(0, rows, body, 0)


def _sc_kernel(t_ref, s_ref, ec1, ec2, ec3, ef1,
               oc1, oc2, oc3, of1,
               sgc1, sgc2, sgc3, sgf1,
               svec_v, idx_v, tstage,
               tc1, tc2, tc3, tf1, sem):
    cid = lax.axis_index("c")
    sid = lax.axis_index("s")

    pltpu.sync_copy(s_ref, svec_v)
    sv = svec_v[...]

    col0 = sid * FILTER_NUM
    pltpu.sync_copy(ef1.at[:, pl.ds(col0, FILTER_NUM)], tstage)
    _sigmoid_block(tstage, sv, N_TASKS, FILTER_NUM)
    pltpu.sync_copy(tstage, sgf1.at[:, pl.ds(col0, FILTER_NUM)])
    for i, (src, dst) in enumerate(((ec1, sgc1), (ec2, sgc2), (ec3, sgc3))):
        @pl.when(sid == i)
        def _():
            pltpu.sync_copy(src, tstage)
            _sigmoid_block(tstage, sv, N_TASKS, FILTER_NUM)
            pltpu.sync_copy(tstage, dst)

    plsc.subcore_barrier()

    for spm, vm in ((sgc1, tc1), (sgc2, tc2), (sgc3, tc3), (sgf1, tf1)):
        pltpu.sync_copy(spm, vm)

    wid = sid * NC + cid
    base = wid * B_PER_W
    pltpu.sync_copy(t_ref.at[pl.ds(base, B_PER_W)], idx_v)

    pairs = ((tc1, oc1), (tc2, oc2), (tc3, oc3), (tf1, of1))

    def chunk_body(c, _):
        v = idx_v[pl.ds(c * L, L)]
        for j in range(L):
            r = v[j]
            i = c * L + j
            for tbl, out in pairs:
                pltpu.async_copy(tbl.at[r], out.at[base + i], sem)
        return 0
    lax.fori_loop(0, B_PER_W // L, chunk_body, 0)

    def drain_body(i, _):
        for tbl, out in pairs:
            pltpu.make_async_copy(tbl.at[0], out.at[base + i], sem).wait()
        return 0
    lax.fori_loop(0, B_PER_W, drain_body, 0)


def _tc_kernel(s_ref, t_ref, tbl2_ref, tbl3_ref, out2_ref, out3_ref):
    i = pl.program_id(0)
    s = s_ref[0]
    tb = t_ref[pl.ds(i * TC_BLOCK, TC_BLOCK)]
    onehot = (tb[:, None] == lax.broadcasted_iota(jnp.int32, (TC_BLOCK, N_TASKS), 1)
              ).astype(jnp.float32)
    sig2 = jax.nn.sigmoid(s * tbl2_ref[...])
    out2_ref[...] = jnp.dot(onehot, sig2, preferred_element_type=jnp.float32)
    sig3 = jax.nn.sigmoid(s * tbl3_ref[...])
    out3_ref[...] = jnp.dot(onehot, sig3, preferred_element_type=jnp.float32)


@jax.jit
def _run(t, svec, ec1, ec2, ec3, efc1, efc2, efc3):
    f32 = jnp.float32
    sc_call = functools.partial(
        pl.kernel,
        out_type=[
            jax.ShapeDtypeStruct((BATCH, FILTER_NUM), f32),
            jax.ShapeDtypeStruct((BATCH, FILTER_NUM), f32),
            jax.ShapeDtypeStruct((BATCH, FILTER_NUM), f32),
            jax.ShapeDtypeStruct((BATCH, NHID), f32),
        ],
        mesh=plsc.VectorSubcoreMesh(core_axis_name="c", subcore_axis_name="s"),
        scratch_types=[
            pltpu.HBM((N_TASKS, FILTER_NUM), f32),
            pltpu.HBM((N_TASKS, FILTER_NUM), f32),
            pltpu.HBM((N_TASKS, FILTER_NUM), f32),
            pltpu.HBM((N_TASKS, NHID), f32),
            pltpu.VMEM((L,), f32),
            pltpu.VMEM((B_PER_W,), jnp.int32),
            pltpu.VMEM((N_TASKS, FILTER_NUM), f32),
            pltpu.VMEM((N_TASKS, FILTER_NUM), f32),
            pltpu.VMEM((N_TASKS, FILTER_NUM), f32),
            pltpu.VMEM((N_TASKS, FILTER_NUM), f32),
            pltpu.VMEM((N_TASKS, NHID), f32),
            pltpu.SemaphoreType.DMA,
        ],
    )(_sc_kernel)
    oc1, oc2, oc3, of1 = sc_call(t, svec, ec1, ec2, ec3, efc1)

    of2, of3 = pl.pallas_call(
        _tc_kernel,
        grid=(BATCH // TC_BLOCK,),
        in_specs=[
            pl.BlockSpec(memory_space=pltpu.SMEM),
            pl.BlockSpec(memory_space=pltpu.VMEM),
            pl.BlockSpec(memory_space=pltpu.VMEM),
            pl.BlockSpec(memory_space=pltpu.VMEM),
        ],
        out_specs=[
            pl.BlockSpec((TC_BLOCK, NHID), lambda i: (i, 0)),
            pl.BlockSpec((TC_BLOCK, NHID), lambda i: (i, 0)),
        ],
        out_shape=[
            jax.ShapeDtypeStruct((BATCH, NHID), f32),
            jax.ShapeDtypeStruct((BATCH, NHID), f32),
        ],
    )(svec[:1], t, efc2, efc3)

    return (oc1, oc2, oc3, of1, of2, of3)


def kernel(t, s, ec1, ec2, ec3, efc1, efc2, efc3):
    t32 = t.astype(jnp.int32)
    svec = jnp.full((L,), s, dtype=jnp.float32)
    return _run(t32, svec, ec1, ec2, ec3, efc1, efc2, efc3)


# speedup vs baseline: 4.6481x; 1.1766x over previous
import functools

import jax
import jax.numpy as jnp
from jax import lax
from jax.experimental import pallas as pl
from jax.experimental.pallas import tpu as pltpu
from jax.experimental.pallas import tpu_sc as plsc

N_TASKS = 10
FILTER_NUM = 128
NHID = 2048
BATCH = 4096

NC = 2
NS = 16
L = 16
NW = NC * NS
B_PER_W = BATCH // NW

TC_BLOCK = 256


def _sigmoid_block(tstage, sv, rows, width):
    def body(r, _):
        for j in range(width // L):
            x = tstage[r, pl.ds(j * L, L)]
            z = sv * x
            ez = jnp.exp(-jnp.abs(z))
            d = 1.0 / (1.0 + ez)
            tstage[r, pl.ds(j * L, L)] = jnp.where(z >= 0, d, ez * d)
        return 0
    lax.fori_loop(0, rows, body, 0)


def _sc_kernel(t_ref, s_ref, ec1, ec2, ec3,
               oc1, oc2, oc3,
               sgc1, sgc2, sgc3,
               svec_v, idx_v, tstage,
               tc1, tc2, tc3, sem):
    cid = lax.axis_index("c")
    sid = lax.axis_index("s")

    pltpu.sync_copy(s_ref, svec_v)
    sv = svec_v[...]

    for i, (src, dst) in enumerate(((ec1, sgc1), (ec2, sgc2), (ec3, sgc3))):
        @pl.when(sid == i)
        def _():
            pltpu.sync_copy(src, tstage)
            _sigmoid_block(tstage, sv, N_TASKS, FILTER_NUM)
            pltpu.sync_copy(tstage, dst)

    plsc.subcore_barrier()

    for spm, vm in ((sgc1, tc1), (sgc2, tc2), (sgc3, tc3)):
        pltpu.sync_copy(spm, vm)

    wid = sid * NC + cid
    base = wid * B_PER_W
    pltpu.sync_copy(t_ref.at[pl.ds(base, B_PER_W)], idx_v)

    pairs = ((tc1, oc1), (tc2, oc2), (tc3, oc3))

    def chunk_body(c, _):
        v = idx_v[pl.ds(c * L, L)]
        for j in range(L):
            r = v[j]
            i = c * L + j
            for tbl, out in pairs:
                pltpu.async_copy(tbl.at[r], out.at[base + i], sem)
        return 0
    lax.fori_loop(0, B_PER_W // L, chunk_body, 0)

    def drain_body(i, _):
        for tbl, out in pairs:
            pltpu.make_async_copy(tbl.at[0], out.at[base + i], sem).wait()
        return 0
    lax.fori_loop(0, B_PER_W, drain_body, 0)


def _tc_kernel(s_ref, t_ref, tbl1_ref, tbl2_ref, tbl3_ref,
               out1_ref, out2_ref, out3_ref):
    i = pl.program_id(0)
    s = s_ref[0]
    tb = t_ref[pl.ds(i * TC_BLOCK, TC_BLOCK)]
    onehot = (tb[:, None] == lax.broadcasted_iota(jnp.int32, (TC_BLOCK, N_TASKS), 1)
              ).astype(jnp.float32)
    for tbl_ref, out_ref in ((tbl1_ref, out1_ref), (tbl2_ref, out2_ref),
                             (tbl3_ref, out3_ref)):
        sig = jax.nn.sigmoid(s * tbl_ref[...])
        out_ref[...] = jnp.dot(onehot, sig, preferred_element_type=jnp.float32)


@jax.jit
def _run(t, svec, ec1, ec2, ec3, efc1, efc2, efc3):
    f32 = jnp.float32
    sc_call = functools.partial(
        pl.kernel,
        out_type=[
            jax.ShapeDtypeStruct((BATCH, FILTER_NUM), f32),
            jax.ShapeDtypeStruct((BATCH, FILTER_NUM), f32),
            jax.ShapeDtypeStruct((BATCH, FILTER_NUM), f32),
        ],
        mesh=plsc.VectorSubcoreMesh(core_axis_name="c", subcore_axis_name="s"),
        scratch_types=[
            pltpu.HBM((N_TASKS, FILTER_NUM), f32),
            pltpu.HBM((N_TASKS, FILTER_NUM), f32),
            pltpu.HBM((N_TASKS, FILTER_NUM), f32),
            pltpu.VMEM((L,), f32),
            pltpu.VMEM((B_PER_W,), jnp.int32),
            pltpu.VMEM((N_TASKS, FILTER_NUM), f32),
            pltpu.VMEM((N_TASKS, FILTER_NUM), f32),
            pltpu.VMEM((N_TASKS, FILTER_NUM), f32),
            pltpu.VMEM((N_TASKS, FILTER_NUM), f32),
            pltpu.SemaphoreType.DMA,
        ],
    )(_sc_kernel)
    oc1, oc2, oc3 = sc_call(t, svec, ec1, ec2, ec3)

    of1, of2, of3 = pl.pallas_call(
        _tc_kernel,
        grid=(BATCH // TC_BLOCK,),
        in_specs=[
            pl.BlockSpec(memory_space=pltpu.SMEM),
            pl.BlockSpec(memory_space=pltpu.VMEM),
            pl.BlockSpec(memory_space=pltpu.VMEM),
            pl.BlockSpec(memory_space=pltpu.VMEM),
            pl.BlockSpec(memory_space=pltpu.VMEM),
        ],
        out_specs=[
            pl.BlockSpec((TC_BLOCK, NHID), lambda i: (i, 0)),
            pl.BlockSpec((TC_BLOCK, NHID), lambda i: (i, 0)),
            pl.BlockSpec((TC_BLOCK, NHID), lambda i: (i, 0)),
        ],
        out_shape=[
            jax.ShapeDtypeStruct((BATCH, NHID), f32),
            jax.ShapeDtypeStruct((BATCH, NHID), f32),
            jax.ShapeDtypeStruct((BATCH, NHID), f32),
        ],
    )(svec[:1], t, efc1, efc2, efc3)

    return (oc1, oc2, oc3, of1, of2, of3)


def kernel(t, s, ec1, ec2, ec3, efc1, efc2, efc3):
    t32 = t.astype(jnp.int32)
    svec = jnp.full((L,), s, dtype=jnp.float32)
    return _run(t32, svec, ec1, ec2, ec3, efc1, efc2, efc3)
